# baseline (speedup 1.0000x reference)
import jax
import jax.numpy as jnp
from jax.experimental import pallas as pl

_BM = 8

_C = (
    0.24987022842600812,
    -0.020347775598867517,
    0.0017623380817153576,
    -0.00011922828226771122,
    5.6748804758556245e-06,
    -1.8126867780952275e-07,
    3.763930553569157e-09,
    -4.8479922268457626e-11,
    3.507893809860871e-13,
    -1.0879359988359795e-15,
)


def _dice_body(inp_ref, tgt_ref, acc_ref):
    i = pl.program_id(0)
    n = inp_ref.shape[1]
    x = jnp.clip(inp_ref[...], -8.0, 8.0)
    t = tgt_ref[...]
    u = x * x
    p = jnp.float32(_C[-1])
    for c in _C[-2::-1]:
        p = p * u + jnp.float32(c)
    s = x * p
    sum_st = jnp.sum(s * t, axis=1)
    sum_s = jnp.sum(s, axis=1)
    sum_t = jnp.sum(t, axis=1)
    num = 2.0 * (sum_st + 0.5 * sum_t)
    den = (sum_s + 0.5 * n) + sum_t
    loss = 1.0 - (num + 1.0) / (den + 1.0)
    ls = jnp.sum(loss).reshape(1, 1)

    @pl.when(i == 0)
    def _init():
        acc_ref[...] = ls

    @pl.when(i > 0)
    def _accum():
        acc_ref[...] += ls


def kernel(inputs, targets, num_boxes):
    n_masks, n_points = inputs.shape
    total = pl.pallas_call(
        _dice_body,
        grid=(n_masks // _BM,),
        in_specs=[
            pl.BlockSpec((_BM, n_points), lambda i: (i, 0)),
            pl.BlockSpec((_BM, n_points), lambda i: (i, 0)),
        ],
        out_specs=pl.BlockSpec((1, 1), lambda i: (0, 0)),
        out_shape=jax.ShapeDtypeStruct((1, 1), jnp.float32),
    )(inputs, targets)
    return total[0, 0] / (num_boxes + 1e-6)


# speedup vs baseline: 1.1299x; 1.1299x over previous
import jax
import jax.numpy as jnp
from jax.experimental import pallas as pl

_BM = 8
_BN = 512

_C = (
    0.24990395925961004,
    -0.020435871793313163,
    0.001795901034182633,
    -0.00012303520659997033,
    5.729155408298089e-06,
    -1.649533378409172e-07,
    2.6158928545591356e-09,
    -1.7372812469973818e-11,
)


def _chunk_sums(x_raw, t):
    x = jnp.clip(x_raw, -6.0, 6.0)
    u = x * x
    p = jnp.float32(_C[-1])
    for c in _C[-2::-1]:
        p = p * u + jnp.float32(c)
    s = x * p
    return s * t, s, t


def _dice_body(inp_ref, tgt_ref, acc_ref):
    i = pl.program_id(0)
    n = inp_ref.shape[1]
    n_full = n // _BN
    tail = n - n_full * _BN

    def step(j, carry):
        a_st, a_s, a_t = carry
        st, s, t = _chunk_sums(
            inp_ref[:, pl.ds(j * _BN, _BN)], tgt_ref[:, pl.ds(j * _BN, _BN)]
        )
        return (a_st + st, a_s + s, a_t + t)

    z = jnp.zeros((_BM, _BN), jnp.float32)
    a_st, a_s, a_t = jax.lax.fori_loop(0, n_full, step, (z, z, z))
    sum_st = jnp.sum(a_st, axis=1)
    sum_s = jnp.sum(a_s, axis=1)
    sum_t = jnp.sum(a_t, axis=1)
    if tail:
        st, s, t = _chunk_sums(
            inp_ref[:, pl.ds(n_full * _BN, tail)],
            tgt_ref[:, pl.ds(n_full * _BN, tail)],
        )
        sum_st += jnp.sum(st, axis=1)
        sum_s += jnp.sum(s, axis=1)
        sum_t += jnp.sum(t, axis=1)

    num = 2.0 * (sum_st + 0.5 * sum_t)
    den = (sum_s + 0.5 * n) + sum_t
    loss = 1.0 - (num + 1.0) / (den + 1.0)
    ls = jnp.sum(loss).reshape(1, 1)

    @pl.when(i == 0)
    def _init():
        acc_ref[...] = ls

    @pl.when(i > 0)
    def _accum():
        acc_ref[...] += ls


def kernel(inputs, targets, num_boxes):
    n_masks, n_points = inputs.shape
    total = pl.pallas_call(
        _dice_body,
        grid=(n_masks // _BM,),
        in_specs=[
            pl.BlockSpec((_BM, n_points), lambda i: (i, 0)),
            pl.BlockSpec((_BM, n_points), lambda i: (i, 0)),
        ],
        out_specs=pl.BlockSpec((1, 1), lambda i: (0, 0)),
        out_shape=jax.ShapeDtypeStruct((1, 1), jnp.float32),
    )(inputs, targets)
    return total[0, 0] / (num_boxes + 1e-6)


# speedup vs baseline: 1.3535x; 1.1979x over previous
import jax
import jax.numpy as jnp
from jax.experimental import pallas as pl

_BM = 8
_BN = 512

_C = (
    0.24990395925961004,
    -0.020435871793313163,
    0.001795901034182633,
    -0.00012303520659997033,
    5.729155408298089e-06,
    -1.649533378409172e-07,
    2.6158928545591356e-09,
    -1.7372812469973818e-11,
)


_UNROLL = 4


def _chunk_sums(x_raw, t):
    c0, c1, c2, c3, c4, c5, c6, c7 = (jnp.float32(c) for c in _C)
    x = jnp.clip(x_raw, -6.0, 6.0)
    u = x * x
    u2 = u * u
    u4 = u2 * u2
    p01 = c0 + c1 * u
    p23 = c2 + c3 * u
    p45 = c4 + c5 * u
    p67 = c6 + c7 * u
    q0 = p01 + u2 * p23
    q1 = p45 + u2 * p67
    p = q0 + u4 * q1
    s = x * p
    return s * t, s, t


def _dice_body(inp_ref, tgt_ref, acc_ref):
    i = pl.program_id(0)
    n = inp_ref.shape[1]
    big = _BN * _UNROLL
    n_full = n // big
    tail_base = n_full * big

    def step(j, carry):
        a_st, a_s, a_t = carry
        base = j * big
        for k in range(_UNROLL):
            st, s, t = _chunk_sums(
                inp_ref[:, pl.ds(base + k * _BN, _BN)],
                tgt_ref[:, pl.ds(base + k * _BN, _BN)],
            )
            a_st, a_s, a_t = a_st + st, a_s + s, a_t + t
        return (a_st, a_s, a_t)

    z = jnp.zeros((_BM, _BN), jnp.float32)
    a_st, a_s, a_t = jax.lax.fori_loop(0, n_full, step, (z, z, z))
    sum_st = jnp.sum(a_st, axis=1)
    sum_s = jnp.sum(a_s, axis=1)
    sum_t = jnp.sum(a_t, axis=1)
    if tail_base < n:
        st, s, t = _chunk_sums(
            inp_ref[:, pl.ds(tail_base, n - tail_base)],
            tgt_ref[:, pl.ds(tail_base, n - tail_base)],
        )
        sum_st += jnp.sum(st, axis=1)
        sum_s += jnp.sum(s, axis=1)
        sum_t += jnp.sum(t, axis=1)

    num = 2.0 * (sum_st + 0.5 * sum_t)
    den = (sum_s + 0.5 * n) + sum_t
    loss = 1.0 - (num + 1.0) / (den + 1.0)
    ls = jnp.sum(loss).reshape(1, 1)

    @pl.when(i == 0)
    def _init():
        acc_ref[...] = ls

    @pl.when(i > 0)
    def _accum():
        acc_ref[...] += ls


def kernel(inputs, targets, num_boxes):
    n_masks, n_points = inputs.shape
    total = pl.pallas_call(
        _dice_body,
        grid=(n_masks // _BM,),
        in_specs=[
            pl.BlockSpec((_BM, n_points), lambda i: (i, 0)),
            pl.BlockSpec((_BM, n_points), lambda i: (i, 0)),
        ],
        out_specs=pl.BlockSpec((1, 1), lambda i: (0, 0)),
        out_shape=jax.ShapeDtypeStruct((1, 1), jnp.float32),
    )(inputs, targets)
    return total[0, 0] / (num_boxes + 1e-6)


# speedup vs baseline: 1.4047x; 1.0378x over previous
import jax
import jax.numpy as jnp
from jax.experimental import pallas as pl

_BM = 8
_BN = 512

_C = (
    0.24990395925961004,
    -0.020435871793313163,
    0.001795901034182633,
    -0.00012303520659997033,
    5.729155408298089e-06,
    -1.649533378409172e-07,
    2.6158928545591356e-09,
    -1.7372812469973818e-11,
)


_UNROLL = 4


def _chunk_sums(x_raw, t):
    c0, c1, c2, c3, c4, c5, c6, c7 = (jnp.float32(c) for c in _C)
    x = jnp.clip(x_raw, -6.0, 6.0)
    u = x * x
    u2 = u * u
    u4 = u2 * u2
    p01 = c0 + c1 * u
    p23 = c2 + c3 * u
    p45 = c4 + c5 * u
    p67 = c6 + c7 * u
    q0 = p01 + u2 * p23
    q1 = p45 + u2 * p67
    p = q0 + u4 * q1
    s = x * p
    return s * t, s, t


def _dice_body(inp_ref, tgt_ref, acc_ref):
    i = pl.program_id(0)
    n = inp_ref.shape[1]
    n_full = n // _BN
    tail_base = n_full * _BN

    z = jnp.zeros((_BM, _BN), jnp.float32)
    a_st, a_s, a_t = z, z, z
    for k in range(n_full):
        st, s, t = _chunk_sums(
            inp_ref[:, k * _BN:(k + 1) * _BN], tgt_ref[:, k * _BN:(k + 1) * _BN]
        )
        a_st, a_s, a_t = a_st + st, a_s + s, a_t + t
    sum_st = jnp.sum(a_st, axis=1)
    sum_s = jnp.sum(a_s, axis=1)
    sum_t = jnp.sum(a_t, axis=1)
    if tail_base < n:
        st, s, t = _chunk_sums(
            inp_ref[:, tail_base:n], tgt_ref[:, tail_base:n]
        )
        sum_st += jnp.sum(st, axis=1)
        sum_s += jnp.sum(s, axis=1)
        sum_t += jnp.sum(t, axis=1)

    num = 2.0 * (sum_st + 0.5 * sum_t)
    den = (sum_s + 0.5 * n) + sum_t
    loss = 1.0 - (num + 1.0) / (den + 1.0)
    ls = jnp.sum(loss).reshape(1, 1)

    @pl.when(i == 0)
    def _init():
        acc_ref[...] = ls

    @pl.when(i > 0)
    def _accum():
        acc_ref[...] += ls


def kernel(inputs, targets, num_boxes):
    n_masks, n_points = inputs.shape
    total = pl.pallas_call(
        _dice_body,
        grid=(n_masks // _BM,),
        in_specs=[
            pl.BlockSpec((_BM, n_points), lambda i: (i, 0)),
            pl.BlockSpec((_BM, n_points), lambda i: (i, 0)),
        ],
        out_specs=pl.BlockSpec((1, 1), lambda i: (0, 0)),
        out_shape=jax.ShapeDtypeStruct((1, 1), jnp.float32),
    )(inputs, targets)
    return total[0, 0] / (num_boxes + 1e-6)


# speedup vs baseline: 1.4696x; 1.0462x over previous
import jax
import jax.numpy as jnp
from jax.experimental import pallas as pl

_BM = 32
_BN = 512

_C = (
    0.24990395925961004,
    -0.020435871793313163,
    0.001795901034182633,
    -0.00012303520659997033,
    5.729155408298089e-06,
    -1.649533378409172e-07,
    2.6158928545591356e-09,
    -1.7372812469973818e-11,
)


_UNROLL = 4


def _chunk_sums(x_raw, t):
    c0, c1, c2, c3, c4, c5, c6, c7 = (jnp.float32(c) for c in _C)
    x = jnp.clip(x_raw, -6.0, 6.0)
    u = x * x
    u2 = u * u
    u4 = u2 * u2
    p01 = c0 + c1 * u
    p23 = c2 + c3 * u
    p45 = c4 + c5 * u
    p67 = c6 + c7 * u
    q0 = p01 + u2 * p23
    q1 = p45 + u2 * p67
    p = q0 + u4 * q1
    s = x * p
    return s * t, s, t


def _dice_body(inp_ref, tgt_ref, acc_ref):
    i = pl.program_id(0)
    n = inp_ref.shape[1]
    n_full = n // _BN
    tail_base = n_full * _BN

    z = jnp.zeros((_BM, _BN), jnp.float32)
    a_st, a_s, a_t = z, z, z
    for k in range(n_full):
        st, s, t = _chunk_sums(
            inp_ref[:, k * _BN:(k + 1) * _BN], tgt_ref[:, k * _BN:(k + 1) * _BN]
        )
        a_st, a_s, a_t = a_st + st, a_s + s, a_t + t
    sum_st = jnp.sum(a_st, axis=1)
    sum_s = jnp.sum(a_s, axis=1)
    sum_t = jnp.sum(a_t, axis=1)
    if tail_base < n:
        st, s, t = _chunk_sums(
            inp_ref[:, tail_base:n], tgt_ref[:, tail_base:n]
        )
        sum_st += jnp.sum(st, axis=1)
        sum_s += jnp.sum(s, axis=1)
        sum_t += jnp.sum(t, axis=1)

    num = 2.0 * (sum_st + 0.5 * sum_t)
    den = (sum_s + 0.5 * n) + sum_t
    loss = 1.0 - (num + 1.0) / (den + 1.0)
    ls = jnp.sum(loss).reshape(1, 1)

    @pl.when(i == 0)
    def _init():
        acc_ref[...] = ls

    @pl.when(i > 0)
    def _accum():
        acc_ref[...] += ls


def kernel(inputs, targets, num_boxes):
    n_masks, n_points = inputs.shape
    total = pl.pallas_call(
        _dice_body,
        grid=(n_masks // _BM,),
        in_specs=[
            pl.BlockSpec((_BM, n_points), lambda i: (i, 0)),
            pl.BlockSpec((_BM, n_points), lambda i: (i, 0)),
        ],
        out_specs=pl.BlockSpec((1, 1), lambda i: (0, 0)),
        out_shape=jax.ShapeDtypeStruct((1, 1), jnp.float32),
    )(inputs, targets)
    return total[0, 0] / (num_boxes + 1e-6)


# speedup vs baseline: 3.9833x; 2.7105x over previous
import jax
import jax.numpy as jnp
from jax.experimental import pallas as pl
from jax.experimental.pallas import tpu as pltpu

_BP = 2000
_CH = 16

_C = (
    0.24990395925961004,
    -0.020435871793313163,
    0.001795901034182633,
    -0.00012303520659997033,
    5.729155408298089e-06,
    -1.649533378409172e-07,
    2.6158928545591356e-09,
    -1.7372812469973818e-11,
)


def _chunk_sums(x_raw, t):
    c0, c1, c2, c3, c4, c5, c6, c7 = (jnp.float32(c) for c in _C)
    x = jnp.clip(x_raw, -6.0, 6.0)
    u = x * x
    u2 = u * u
    u4 = u2 * u2
    p01 = c0 + c1 * u
    p23 = c2 + c3 * u
    p45 = c4 + c5 * u
    p67 = c6 + c7 * u
    q0 = p01 + u2 * p23
    q1 = p45 + u2 * p67
    p = q0 + u4 * q1
    s = x * p
    return s * t, s, t


def _dice_body(inp_ref, tgt_ref, out_ref, a_st_ref, a_s_ref, a_t_ref):
    i = pl.program_id(0)
    n_steps = pl.num_programs(0)
    bp, m = inp_ref.shape

    z = jnp.zeros((_CH, m), jnp.float32)
    a_st, a_s, a_t = z, z, z
    for k in range(bp // _CH):
        st, s, t = _chunk_sums(
            inp_ref[k * _CH:(k + 1) * _CH, :], tgt_ref[k * _CH:(k + 1) * _CH, :]
        )
        a_st, a_s, a_t = a_st + st, a_s + s, a_t + t

    @pl.when(i == 0)
    def _init():
        a_st_ref[...] = a_st
        a_s_ref[...] = a_s
        a_t_ref[...] = a_t

    @pl.when(i > 0)
    def _accum():
        a_st_ref[...] += a_st
        a_s_ref[...] += a_s
        a_t_ref[...] += a_t

    @pl.when(i == n_steps - 1)
    def _final():
        n = bp * n_steps
        sum_st = jnp.sum(a_st_ref[...], axis=0)
        sum_s = jnp.sum(a_s_ref[...], axis=0)
        sum_t = jnp.sum(a_t_ref[...], axis=0)
        num = 2.0 * (sum_st + 0.5 * sum_t)
        den = (sum_s + 0.5 * n) + sum_t
        loss = 1.0 - (num + 1.0) / (den + 1.0)
        out_ref[...] = jnp.sum(loss).reshape(1, 1)


def kernel(inputs, targets, num_boxes):
    n_masks, n_points = inputs.shape
    xt = inputs.T
    tt = targets.T
    total = pl.pallas_call(
        _dice_body,
        grid=(n_points // _BP,),
        in_specs=[
            pl.BlockSpec((_BP, n_masks), lambda i: (i, 0)),
            pl.BlockSpec((_BP, n_masks), lambda i: (i, 0)),
        ],
        out_specs=pl.BlockSpec((1, 1), lambda i: (0, 0)),
        out_shape=jax.ShapeDtypeStruct((1, 1), jnp.float32),
        scratch_shapes=[pltpu.VMEM((_CH, n_masks), jnp.float32)] * 3,
    )(xt, tt)
    return total[0, 0] / (num_boxes + 1e-6)


# speedup vs baseline: 5.0356x; 1.2642x over previous
import jax
import jax.numpy as jnp
from jax.experimental import pallas as pl
from jax.experimental.pallas import tpu as pltpu

_BP = 2000
_CH = 16

_C = (
    0.24990395925961004,
    -0.020435871793313163,
    0.001795901034182633,
    -0.00012303520659997033,
    5.729155408298089e-06,
    -1.649533378409172e-07,
    2.6158928545591356e-09,
    -1.7372812469973818e-11,
)


def _chunk_sums(x_raw, t):
    s = jax.nn.sigmoid(x_raw)
    return s * t, s, t


def _dice_body(inp_ref, tgt_ref, out_ref, a_st_ref, a_s_ref, a_t_ref):
    i = pl.program_id(0)
    n_steps = pl.num_programs(0)
    bp, m = inp_ref.shape

    z = jnp.zeros((_CH, m), jnp.float32)
    a_st, a_s, a_t = z, z, z
    for k in range(bp // _CH):
        st, s, t = _chunk_sums(
            inp_ref[k * _CH:(k + 1) * _CH, :], tgt_ref[k * _CH:(k + 1) * _CH, :]
        )
        a_st, a_s, a_t = a_st + st, a_s + s, a_t + t

    @pl.when(i == 0)
    def _init():
        a_st_ref[...] = a_st
        a_s_ref[...] = a_s
        a_t_ref[...] = a_t

    @pl.when(i > 0)
    def _accum():
        a_st_ref[...] += a_st
        a_s_ref[...] += a_s
        a_t_ref[...] += a_t

    @pl.when(i == n_steps - 1)
    def _final():
        sum_st = jnp.sum(a_st_ref[...], axis=0)
        sum_s = jnp.sum(a_s_ref[...], axis=0)
        sum_t = jnp.sum(a_t_ref[...], axis=0)
        num = 2.0 * sum_st
        den = sum_s + sum_t
        loss = 1.0 - (num + 1.0) / (den + 1.0)
        out_ref[...] = jnp.sum(loss).reshape(1, 1)


def kernel(inputs, targets, num_boxes):
    n_masks, n_points = inputs.shape
    xt = inputs.T
    tt = targets.T
    total = pl.pallas_call(
        _dice_body,
        grid=(n_points // _BP,),
        in_specs=[
            pl.BlockSpec((_BP, n_masks), lambda i: (i, 0)),
            pl.BlockSpec((_BP, n_masks), lambda i: (i, 0)),
        ],
        out_specs=pl.BlockSpec((1, 1), lambda i: (0, 0)),
        out_shape=jax.ShapeDtypeStruct((1, 1), jnp.float32),
        scratch_shapes=[pltpu.VMEM((_CH, n_masks), jnp.float32)] * 3,
    )(xt, tt)
    return total[0, 0] / (num_boxes + 1e-6)


# speedup vs baseline: 5.5720x; 1.1065x over previous
import jax
import jax.numpy as jnp
from jax.experimental import pallas as pl
from jax.experimental.pallas import tpu as pltpu

_BP = 5000
_CH = 40

_C = (
    0.24990395925961004,
    -0.020435871793313163,
    0.001795901034182633,
    -0.00012303520659997033,
    5.729155408298089e-06,
    -1.649533378409172e-07,
    2.6158928545591356e-09,
    -1.7372812469973818e-11,
)


def _chunk_sums(x_raw, t):
    s = jax.nn.sigmoid(x_raw)
    return s * t, s, t


def _dice_body(inp_ref, tgt_ref, out_ref, a_st_ref, a_s_ref, a_t_ref):
    i = pl.program_id(0)
    n_steps = pl.num_programs(0)
    bp, m = inp_ref.shape

    z = jnp.zeros((_CH, m), jnp.float32)
    a_st, a_s, a_t = z, z, z
    for k in range(bp // _CH):
        st, s, t = _chunk_sums(
            inp_ref[k * _CH:(k + 1) * _CH, :], tgt_ref[k * _CH:(k + 1) * _CH, :]
        )
        a_st, a_s, a_t = a_st + st, a_s + s, a_t + t

    @pl.when(i == 0)
    def _init():
        a_st_ref[...] = a_st
        a_s_ref[...] = a_s
        a_t_ref[...] = a_t

    @pl.when(i > 0)
    def _accum():
        a_st_ref[...] += a_st
        a_s_ref[...] += a_s
        a_t_ref[...] += a_t

    @pl.when(i == n_steps - 1)
    def _final():
        sum_st = jnp.sum(a_st_ref[...], axis=0)
        sum_s = jnp.sum(a_s_ref[...], axis=0)
        sum_t = jnp.sum(a_t_ref[...], axis=0)
        num = 2.0 * sum_st
        den = sum_s + sum_t
        loss = 1.0 - (num + 1.0) / (den + 1.0)
        out_ref[...] = jnp.sum(loss).reshape(1, 1)


def kernel(inputs, targets, num_boxes):
    n_masks, n_points = inputs.shape
    xt = inputs.T
    tt = targets.T
    total = pl.pallas_call(
        _dice_body,
        grid=(n_points // _BP,),
        in_specs=[
            pl.BlockSpec((_BP, n_masks), lambda i: (i, 0)),
            pl.BlockSpec((_BP, n_masks), lambda i: (i, 0)),
        ],
        out_specs=pl.BlockSpec((1, 1), lambda i: (0, 0)),
        out_shape=jax.ShapeDtypeStruct((1, 1), jnp.float32),
        scratch_shapes=[pltpu.VMEM((_CH, n_masks), jnp.float32)] * 3,
    )(xt, tt)
    return total[0, 0] / (num_boxes + 1e-6)
